# baseline (speedup 1.0000x reference)
import functools

import jax
import jax.numpy as jnp
from jax import lax
from jax.experimental import pallas as pl
from jax.experimental.pallas import tpu as pltpu
from jax.experimental.pallas import tpu_sc as plsc

NUM_SEGMENTS = 16
D_MODEL = 1024

_INFO = plsc.get_sparse_core_info()
_NC, _NS = _INFO.num_cores, _INFO.num_subcores
_NW = _NC * _NS

_B = 4 * 4096
_BPW = _B // _NW
_C = 32
_NCHUNK = _BPW // _C
_NBUF = 3


_SL = 8
_LN = D_MODEL // _SL


@functools.partial(
    pl.kernel,
    mesh=plsc.VectorSubcoreMesh(core_axis_name="c", subcore_axis_name="s"),
    out_type=jax.ShapeDtypeStruct((_B, _SL, _LN), jnp.float32),
    scratch_types=[
        pltpu.VMEM((_BPW,), jnp.int32),
        pltpu.VMEM((_NBUF, _C, _SL, _LN), jnp.float32),
        pltpu.SemaphoreType.DMA((_NBUF,)),
        pltpu.SemaphoreType.DMA((_NBUF,)),
    ],
)
def _sc_lookup(seg_hbm, table_hbm, out_hbm, idx_v, bufs, gsem, wsem):
    wid = lax.axis_index("s") * _NC + lax.axis_index("c")
    base = wid * _BPW
    pltpu.sync_copy(seg_hbm.at[pl.ds(base, _BPW)], idx_v)

    def gather(chunk, b):
        return pltpu.async_copy(
            table_hbm.at[idx_v.at[pl.ds(chunk * _C, _C)]],
            bufs.at[b],
            gsem.at[b],
        )

    def write(chunk, b):
        return pltpu.async_copy(
            bufs.at[b],
            out_hbm.at[pl.ds(base + chunk * _C, _C)],
            wsem.at[b],
        )

    gh = [None] * _NBUF
    wh = [None] * _NBUF
    for b in range(_NBUF):
        gh[b] = gather(b, b)
    for c in range(_NCHUNK):
        b = c % _NBUF
        if c >= _NBUF:
            wh[b].wait()
            gh[b] = gather(c, b)
        gh[b].wait()
        wh[b] = write(c, b)
    for c in range(_NCHUNK - _NBUF, _NCHUNK):
        wh[c % _NBUF].wait()


def kernel(segment_ids, table):
    seg_flat = segment_ids.reshape(-1).astype(jnp.int32)
    table3 = table.reshape(NUM_SEGMENTS, _SL, _LN)
    out = _sc_lookup(seg_flat, table3)
    return out.reshape(segment_ids.shape + (D_MODEL,))


# speedup vs baseline: 1.3693x; 1.3693x over previous
import functools

import jax
import jax.numpy as jnp
from jax import lax
from jax.experimental import pallas as pl
from jax.experimental.pallas import tpu as pltpu
from jax.experimental.pallas import tpu_sc as plsc

NUM_SEGMENTS = 16
D_MODEL = 1024

_INFO = plsc.get_sparse_core_info()
_NC, _NS, _L = _INFO.num_cores, _INFO.num_subcores, _INFO.num_lanes
_NW = _NC * _NS

_B = 4 * 4096
_BPW = _B // _NW
_C = 32
_NCHUNK = _BPW // _C
_NBUF = 3


@functools.partial(
    pl.kernel,
    mesh=plsc.VectorSubcoreMesh(core_axis_name="c", subcore_axis_name="s"),
    out_type=jax.ShapeDtypeStruct((_B, D_MODEL), jnp.float32),
    scratch_types=[
        pltpu.VMEM((NUM_SEGMENTS, D_MODEL), jnp.float32),
        pltpu.VMEM((_BPW + _L,), jnp.int32),
        pltpu.VMEM((_NBUF * _C, D_MODEL), jnp.float32),
        pltpu.SemaphoreType.DMA((_NBUF,)),
    ],
)
def _sc_lookup(seg_hbm, table_hbm, out_hbm, table_v, idx_v, bufs, wsem):
    wid = lax.axis_index("s") * _NC + lax.axis_index("c")
    base = wid * _BPW
    pltpu.sync_copy(table_hbm, table_v)
    pltpu.sync_copy(seg_hbm.at[pl.ds(base, _BPW)], idx_v.at[pl.ds(0, _BPW)])

    wh = [None] * _NBUF
    for c in range(_NCHUNK):
        b = c % _NBUF
        if c >= _NBUF:
            wh[b].wait()

        def row_body(r, _, b=b, c=c):
            seg = idx_v[pl.ds(c * _C + r, _L)][0]
            dst = b * _C + r
            for j in range(D_MODEL // _L):
                bufs[dst, pl.ds(j * _L, _L)] = table_v[seg, pl.ds(j * _L, _L)]
            return 0

        lax.fori_loop(0, _C, row_body, 0)
        wh[b] = pltpu.async_copy(
            bufs.at[pl.ds(b * _C, _C)],
            out_hbm.at[pl.ds(base + c * _C, _C)],
            wsem.at[b],
        )
    for c in range(_NCHUNK - _NBUF, _NCHUNK):
        wh[c % _NBUF].wait()


def kernel(segment_ids, table):
    seg_flat = segment_ids.reshape(-1).astype(jnp.int32)
    out = _sc_lookup(seg_flat, table)
    return out.reshape(segment_ids.shape + (D_MODEL,))


# speedup vs baseline: 4.1036x; 2.9969x over previous
import functools

import jax
import jax.numpy as jnp
from jax import lax
from jax.experimental import pallas as pl
from jax.experimental.pallas import tpu as pltpu
from jax.experimental.pallas import tpu_sc as plsc

NUM_SEGMENTS = 16
D_MODEL = 1024

_INFO = plsc.get_sparse_core_info()
_NC, _NS, _L = _INFO.num_cores, _INFO.num_subcores, _INFO.num_lanes
_NW = _NC * _NS

_B = 4 * 4096
_BPW = _B // _NW
_C = 32
_NCHUNK = _BPW // _C
_NBUF = 3


@functools.partial(
    pl.kernel,
    mesh=plsc.VectorSubcoreMesh(core_axis_name="c", subcore_axis_name="s"),
    out_type=jax.ShapeDtypeStruct((_B, D_MODEL), jnp.float32),
    scratch_types=[
        pltpu.VMEM_SHARED((NUM_SEGMENTS, D_MODEL), jnp.float32),
        pltpu.VMEM((_BPW + _L,), jnp.int32),
        pltpu.VMEM((_NBUF * _C, D_MODEL), jnp.float32),
        pltpu.SemaphoreType.DMA((_NBUF,)),
        pltpu.SemaphoreType.DMA((_NBUF,)),
    ],
)
def _sc_lookup(seg_hbm, table_hbm, out_hbm, table_sp, idx_v, bufs, gsem, wsem):
    cid = lax.axis_index("c")
    sid = lax.axis_index("s")
    wid = sid * _NC + cid
    base = wid * _BPW

    @pl.when(sid == 0)
    def _stage_table():
        pltpu.sync_copy(table_hbm, table_sp)

    pltpu.sync_copy(seg_hbm.at[pl.ds(base, _BPW)], idx_v.at[pl.ds(0, _BPW)])
    plsc.subcore_barrier()

    def gather(chunk, b):
        def row_body(r, _):
            seg = idx_v[pl.ds(chunk * _C + r, _L)][0]
            pltpu.async_copy(table_sp.at[seg], bufs.at[b * _C + r], gsem.at[b])
            return 0

        lax.fori_loop(0, _C, row_body, 0)

    def gather_wait(b):
        pltpu.make_async_copy(
            out_hbm.at[pl.ds(0, _C)],
            bufs.at[pl.ds(b * _C, _C)],
            gsem.at[b],
        ).wait()

    def write(chunk, b):
        return pltpu.async_copy(
            bufs.at[pl.ds(b * _C, _C)],
            out_hbm.at[pl.ds(base + chunk * _C, _C)],
            wsem.at[b],
        )

    wh = [None] * _NBUF
    gather(0, 0)
    for c in range(_NCHUNK):
        b = c % _NBUF
        nc = c + 1
        if nc < _NCHUNK:
            nb = nc % _NBUF
            if wh[nb] is not None:
                wh[nb].wait()
                wh[nb] = None
            gather(nc, nb)
        gather_wait(b)
        wh[b] = write(c, b)
    for b in range(_NBUF):
        if wh[b] is not None:
            wh[b].wait()


def kernel(segment_ids, table):
    seg_flat = segment_ids.reshape(-1).astype(jnp.int32)
    out = _sc_lookup(seg_flat, table)
    return out.reshape(segment_ids.shape + (D_MODEL,))


# speedup vs baseline: 4.2502x; 1.0357x over previous
import functools

import jax
import jax.numpy as jnp
from jax import lax
from jax.experimental import pallas as pl
from jax.experimental.pallas import tpu as pltpu
from jax.experimental.pallas import tpu_sc as plsc

NUM_SEGMENTS = 16
D_MODEL = 1024

_INFO = plsc.get_sparse_core_info()
_NC, _NS, _L = _INFO.num_cores, _INFO.num_subcores, _INFO.num_lanes
_NW = _NC * _NS

_B = 4 * 4096
_BPW = _B // _NW


@functools.partial(
    pl.kernel,
    mesh=plsc.VectorSubcoreMesh(core_axis_name="c", subcore_axis_name="s"),
    out_type=jax.ShapeDtypeStruct((_B, D_MODEL), jnp.float32),
    scratch_types=[
        pltpu.VMEM((NUM_SEGMENTS, D_MODEL), jnp.float32),
        pltpu.VMEM((_BPW + _L,), jnp.int32),
        pltpu.SemaphoreType.DMA,
    ],
)
def _sc_lookup(seg_hbm, table_hbm, out_hbm, table_v, idx_v, wsem):
    wid = lax.axis_index("s") * _NC + lax.axis_index("c")
    base = wid * _BPW
    pltpu.sync_copy(table_hbm, table_v)
    pltpu.sync_copy(seg_hbm.at[pl.ds(base, _BPW)], idx_v.at[pl.ds(0, _BPW)])

    def issue_row(r, _):
        seg = idx_v[pl.ds(r, _L)][0]
        pltpu.async_copy(table_v.at[seg], out_hbm.at[base + r], wsem)
        return 0

    lax.fori_loop(0, _BPW, issue_row, 0)

    def drain_row(r, _):
        pltpu.make_async_copy(table_v.at[0], out_hbm.at[base + r], wsem).wait()
        return 0

    lax.fori_loop(0, _BPW, drain_row, 0)


def kernel(segment_ids, table):
    seg_flat = segment_ids.reshape(-1).astype(jnp.int32)
    out = _sc_lookup(seg_flat, table)
    return out.reshape(segment_ids.shape + (D_MODEL,))


# speedup vs baseline: 4.3211x; 1.0167x over previous
import functools

import jax
import jax.numpy as jnp
from jax import lax
from jax.experimental import pallas as pl
from jax.experimental.pallas import tpu as pltpu
from jax.experimental.pallas import tpu_sc as plsc

NUM_SEGMENTS = 16
D_MODEL = 1024

_INFO = plsc.get_sparse_core_info()
_NC, _NS, _L = _INFO.num_cores, _INFO.num_subcores, _INFO.num_lanes
_NW = _NC * _NS

_B = 4 * 4096
_BPW = _B // _NW
_NG = _BPW // _L


@functools.partial(
    pl.kernel,
    mesh=plsc.VectorSubcoreMesh(core_axis_name="c", subcore_axis_name="s"),
    out_type=jax.ShapeDtypeStruct((_B, D_MODEL), jnp.float32),
    scratch_types=[
        pltpu.VMEM((NUM_SEGMENTS, D_MODEL), jnp.float32),
        pltpu.VMEM((_BPW,), jnp.int32),
        pltpu.SemaphoreType.DMA,
    ],
)
def _sc_lookup(seg_hbm, table_hbm, out_hbm, table_v, idx_v, wsem):
    wid = lax.axis_index("s") * _NC + lax.axis_index("c")
    base = wid * _BPW
    pltpu.sync_copy(table_hbm, table_v)
    pltpu.sync_copy(seg_hbm.at[pl.ds(base, _BPW)], idx_v)

    def issue_group(g, _):
        idxs = idx_v[pl.ds(g * _L, _L)]
        row = base + g * _L
        for l in range(_L):
            pltpu.async_copy(table_v.at[idxs[l]], out_hbm.at[row + l], wsem)
        return 0

    lax.fori_loop(0, _NG, issue_group, 0)

    def drain_group(g, _):
        pltpu.make_async_copy(
            table_v, out_hbm.at[pl.ds(base + g * _L, _L)], wsem
        ).wait()
        return 0

    lax.fori_loop(0, _NG, drain_group, 0)


def kernel(segment_ids, table):
    seg_flat = segment_ids.reshape(-1).astype(jnp.int32)
    out = _sc_lookup(seg_flat, table)
    return out.reshape(segment_ids.shape + (D_MODEL,))
